# baseline (speedup 1.0000x reference)
import functools

import jax
import jax.numpy as jnp
from jax import lax
from jax.experimental import pallas as pl
from jax.experimental.pallas import tpu as pltpu
from jax.experimental.pallas import tpu_sc as plsc

NW = 32
ROWS_PER_STEP = 1
NBUF = 4


def kernel(cell_index, cell_means):
    B = cell_index.shape[0]
    V, D = cell_means.shape
    b_per_w = B // NW
    n_steps = b_per_w // ROWS_PER_STEP

    idx = cell_index.astype(jnp.int32).reshape(NW, n_steps, ROWS_PER_STEP)

    mesh = plsc.VectorSubcoreMesh(core_axis_name="c", subcore_axis_name="s")

    @functools.partial(
        pl.kernel,
        mesh=mesh,
        out_type=jax.ShapeDtypeStruct((B, D), jnp.float32),
        compiler_params=pltpu.CompilerParams(use_tc_tiling_on_sc=False),
        scratch_types=[
            pltpu.VMEM((n_steps, ROWS_PER_STEP), jnp.int32),
            *[pltpu.VMEM((ROWS_PER_STEP, D), jnp.float32) for _ in range(NBUF)],
            *[pltpu.SemaphoreType.DMA for _ in range(2 * NBUF)],
        ],
    )
    def gather_kernel(idx_hbm, table_hbm, out_hbm, idx_v, *rest):
        bufs = rest[:NBUF]
        gsems = rest[NBUF : 2 * NBUF]
        wsems = rest[2 * NBUF :]
        wid = lax.axis_index("s") * 2 + lax.axis_index("c")
        base = wid * b_per_w
        pltpu.sync_copy(idx_hbm.at[wid], idx_v)

        def g_start(step, b):
            pltpu.async_copy(table_hbm.at[idx_v.at[step]], bufs[b], gsems[b])

        def g_wait(step, b):
            pltpu.make_async_copy(
                table_hbm.at[idx_v.at[step]], bufs[b], gsems[b]
            ).wait()

        def out_at(step):
            return out_hbm.at[pl.ds(base + step * ROWS_PER_STEP, ROWS_PER_STEP)]

        def w_start(step, b):
            pltpu.async_copy(bufs[b], out_at(step), wsems[b])

        def w_wait(step, b):
            pltpu.make_async_copy(bufs[b], out_at(step), wsems[b]).wait()

        for j in range(NBUF):
            g_start(j, j)

        LAG = NBUF // 2

        def body(i4, carry):
            for b in range(NBUF):
                step = i4 * NBUF + b
                g_wait(step, b)
                w_start(step, b)

                @pl.when(step >= LAG)
                def _():
                    j = step - LAG
                    jb = (b - LAG) % NBUF
                    w_wait(j, jb)

                    @pl.when(step + NBUF - LAG < n_steps)
                    def _():
                        g_start(j + NBUF, jb)

            return carry

        lax.fori_loop(0, n_steps // NBUF, body, 0)

        for j in range(n_steps - LAG, n_steps):
            w_wait(j, j % NBUF)

    return gather_kernel(idx, cell_means)


# speedup vs baseline: 1.5238x; 1.5238x over previous
import functools

import jax
import jax.numpy as jnp
from jax import lax
from jax.experimental import pallas as pl
from jax.experimental.pallas import tpu as pltpu
from jax.experimental.pallas import tpu_sc as plsc

NW = 32
NBUF = 4
LANES = 128


def _sc_gather_main(idx2d, cell_means, B, D, DM):
    b_per_w = B // NW

    mesh = plsc.VectorSubcoreMesh(core_axis_name="c", subcore_axis_name="s")

    @functools.partial(
        pl.kernel,
        mesh=mesh,
        out_type=jax.ShapeDtypeStruct((B, D), jnp.float32),
        scratch_types=[
            pltpu.VMEM((NW, b_per_w), jnp.int32),
            *[pltpu.VMEM((1, DM), jnp.float32) for _ in range(NBUF)],
            *[pltpu.SemaphoreType.DMA for _ in range(2 * NBUF)],
        ],
    )
    def gather_kernel(idx_hbm, table_hbm, out_hbm, idx_v, *rest):
        bufs = rest[:NBUF]
        gsems = rest[NBUF : 2 * NBUF]
        wsems = rest[2 * NBUF :]
        wid = lax.axis_index("s") * 2 + lax.axis_index("c")
        base = wid * b_per_w
        pltpu.sync_copy(idx_hbm, idx_v)

        def g_start(step, b):
            pltpu.async_copy(
                table_hbm.at[idx_v.at[wid, pl.ds(step, 1)], pl.ds(0, DM)],
                bufs[b],
                gsems[b],
            )

        def g_wait(step, b):
            pltpu.make_async_copy(
                table_hbm.at[idx_v.at[wid, pl.ds(step, 1)], pl.ds(0, DM)],
                bufs[b],
                gsems[b],
            ).wait()

        def out_at(step):
            return out_hbm.at[pl.ds(base + step, 1), pl.ds(0, DM)]

        def w_start(step, b):
            pltpu.async_copy(bufs[b], out_at(step), wsems[b])

        def w_wait(step, b):
            pltpu.make_async_copy(bufs[b], out_at(step), wsems[b]).wait()

        for j in range(NBUF):
            g_start(j, j)

        LAG = NBUF // 2

        def body(i4, carry):
            for b in range(NBUF):
                step = i4 * NBUF + b
                g_wait(step, b)
                w_start(step, b)

                @pl.when(step >= LAG)
                def _():
                    j = step - LAG
                    jb = (b - LAG) % NBUF
                    w_wait(j, jb)

                    @pl.when(step + NBUF - LAG < b_per_w)
                    def _():
                        g_start(j + NBUF, jb)

            return carry

        lax.fori_loop(0, b_per_w // NBUF, body, 0)

        for j in range(b_per_w - LAG, b_per_w):
            w_wait(j, j % NBUF)

    return gather_kernel(idx2d, cell_means)


def _tc_gather_tail(idx, cell_means, out_main, B, V, D, DM):
    DT = LANES

    def tail_kernel(idx_ref, tab_ref, _, o_ref):
        ids = idx_ref[:, 0]
        onehot = (
            ids[:, None] == lax.broadcasted_iota(jnp.int32, (B, V), 1)
        ).astype(jnp.float32)
        o_ref[...] = jnp.dot(
            onehot, tab_ref[...], preferred_element_type=jnp.float32
        )

    return pl.pallas_call(
        tail_kernel,
        grid=(1,),
        in_specs=[
            pl.BlockSpec((B, 1), lambda g: (0, 0)),
            pl.BlockSpec((V, DT), lambda g: (0, DM // DT)),
            pl.BlockSpec(memory_space=pl.ANY),
        ],
        out_specs=pl.BlockSpec((B, DT), lambda g: (0, DM // DT)),
        out_shape=jax.ShapeDtypeStruct((B, D), jnp.float32),
        input_output_aliases={2: 0},
    )(idx.reshape(B, 1), cell_means, out_main)


def kernel(cell_index, cell_means):
    B = cell_index.shape[0]
    V, D = cell_means.shape
    DM = (D // LANES) * LANES

    idx = cell_index.astype(jnp.int32)
    idx2d = idx.reshape(NW, B // NW)

    out_main = _sc_gather_main(idx2d, cell_means, B, D, DM)
    if DM == D:
        return out_main
    return _tc_gather_tail(idx, cell_means, out_main, B, V, D, DM)
